# baseline (speedup 1.0000x reference)
import functools

import jax
import jax.numpy as jnp
from jax import lax
from jax.experimental import pallas as pl
from jax.experimental.pallas import tpu as pltpu
from jax.experimental.pallas import tpu_sc as plsc

_DIM = 2048
_BATCH = 128
_STAGES = 3
_NC = 2
_NW = 16
_BPW = _BATCH // _NW

_mesh = plsc.VectorSubcoreMesh(core_axis_name="c", subcore_axis_name="s")


@functools.partial(
    pl.kernel,
    mesh=_mesh,
    out_type=jax.ShapeDtypeStruct((_BATCH, 1, _DIM), jnp.float32),
    scratch_types=[
        pltpu.VMEM((_BPW,), jnp.int32),
        pltpu.VMEM_SHARED((_STAGES, 1, _DIM), jnp.float32),
        pltpu.VMEM((_BPW, 1, _DIM), jnp.float32),
        pltpu.SemaphoreType.DMA,
        pltpu.SemaphoreType.DMA,
    ],
)
def _embed(idx_hbm, table_hbm, out_hbm, idx_v, table_sh, rows_v, sem_t, sem_g):
    wid = lax.axis_index("s") * _NC + lax.axis_index("c")

    @pl.when(wid < _NW)
    def _():
        base = wid * _BPW
        cp_t = pltpu.async_copy(table_hbm, table_sh, sem_t)
        pltpu.sync_copy(idx_hbm.at[pl.ds(base, _BPW)], idx_v)
        cp_t.wait()
        pltpu.async_copy(table_sh.at[idx_v], rows_v, sem_g).wait()
        pltpu.sync_copy(rows_v, out_hbm.at[pl.ds(base, _BPW)])


def kernel(stage_id, weight):
    return _embed(stage_id.astype(jnp.int32), weight.reshape(_STAGES, 1, _DIM))


# speedup vs baseline: 1.0079x; 1.0079x over previous
import functools

import jax
import jax.numpy as jnp
from jax import lax
from jax.experimental import pallas as pl
from jax.experimental.pallas import tpu as pltpu
from jax.experimental.pallas import tpu_sc as plsc

_DIM = 2048
_BATCH = 128
_STAGES = 3
_NC = 2
_NW = 16
_BPW = _BATCH // _NW

_mesh = plsc.VectorSubcoreMesh(core_axis_name="c", subcore_axis_name="s")


@functools.partial(
    pl.kernel,
    mesh=_mesh,
    out_type=jax.ShapeDtypeStruct((_BATCH, 1, _DIM), jnp.float32),
    scratch_types=[
        pltpu.VMEM((_BPW,), jnp.int32),
        pltpu.VMEM_SHARED((_STAGES, 1, _DIM), jnp.float32),
        pltpu.VMEM((_BPW, 1, _DIM), jnp.float32),
        pltpu.SemaphoreType.DMA,
        pltpu.SemaphoreType.DMA,
    ],
)
def _embed(idx_hbm, table_hbm, out_hbm, idx_v, table_sh, rows_v, sem_t, sem_g):
    wid = lax.axis_index("s") * _NC + lax.axis_index("c")

    sid = lax.axis_index("s")

    @pl.when(sid == 8)
    def _():
        pltpu.async_copy(table_hbm, table_sh, sem_t).wait()

    @pl.when(wid < _NW)
    def _():
        pltpu.sync_copy(idx_hbm.at[pl.ds(wid * _BPW, _BPW)], idx_v)

    plsc.subcore_barrier()

    @pl.when(wid < _NW)
    def _():
        base = wid * _BPW
        pltpu.async_copy(table_sh.at[idx_v], rows_v, sem_g).wait()
        pltpu.sync_copy(rows_v, out_hbm.at[pl.ds(base, _BPW)])


def kernel(stage_id, weight):
    return _embed(stage_id.astype(jnp.int32), weight.reshape(_STAGES, 1, _DIM))
